# baseline (speedup 1.0000x reference)
import functools

import jax
import jax.numpy as jnp
from jax import lax
from jax.experimental import pallas as pl
from jax.experimental.pallas import tpu as pltpu
from jax.experimental.pallas import tpu_sc as plsc

_TILE = 8
_BIG = 2**30
_NW = 32
_QPW = 128
_QB_TC = 2000


def _match_block_kernel(onehot_ref, gtrep_ref, logits_ref, pboxes_ref,
                        min_ref, idx_ref, *, qb: int, g: int):
    qi = pl.program_id(1)

    l = logits_ref[0]
    m = jnp.max(l, axis=-1, keepdims=True)
    e = jnp.exp(l - m)
    s = jnp.sum(e, axis=-1, keepdims=True)

    eg = jnp.dot(e, onehot_ref[0], preferred_element_type=jnp.float32,
                 precision=jax.lax.Precision.HIGHEST)

    pb = pboxes_ref[0]

    gt = gtrep_ref[0]
    bgcx, bgcy, bgw, bgh = (gt[0:8], gt[8:16], gt[16:24], gt[24:32])
    bgx1, bgy1, bgx2, bgy2 = (gt[32:40], gt[40:48], gt[48:56], gt[56:64])
    bgarea = gt[64:72]

    ntiles = qb // _TILE
    accs = []
    for k in range(4):
        accs.append([jnp.full((_TILE, g), jnp.inf, jnp.float32),
                     jnp.zeros((_TILE, g), jnp.int32)])
    base_iota = jax.lax.broadcasted_iota(jnp.int32, (_TILE, g), 0)

    for i in range(ntiles):
        sl = slice(i * _TILE, (i + 1) * _TILE)
        pbt = pb[sl, :]
        bcx = jnp.broadcast_to(pbt[:, 0:1], (_TILE, g))
        bcy = jnp.broadcast_to(pbt[:, 1:2], (_TILE, g))
        bw = jnp.broadcast_to(pbt[:, 2:3], (_TILE, g))
        bh = jnp.broadcast_to(pbt[:, 3:4], (_TILE, g))
        bs = jnp.broadcast_to(s[sl, :], (_TILE, g))
        prob = eg[sl, :] / bs

        hw = 0.5 * bw
        hh = 0.5 * bh
        px1 = bcx - hw
        py1 = bcy - hh
        px2 = bcx + hw
        py2 = bcy + hh
        parea = (px2 - px1) * (py2 - py1)

        bbox = (jnp.abs(bcx - bgcx) + jnp.abs(bcy - bgcy)
                + jnp.abs(bw - bgw) + jnp.abs(bh - bgh))

        iw = jnp.maximum(jnp.minimum(px2, bgx2) - jnp.maximum(px1, bgx1), 0.0)
        ih = jnp.maximum(jnp.minimum(py2, bgy2) - jnp.maximum(py1, bgy1), 0.0)
        inter = iw * ih
        union = parea + bgarea - inter
        iou = inter / (union + 1e-8)
        ew = jnp.maximum(px2, bgx2) - jnp.minimum(px1, bgx1)
        eh = jnp.maximum(py2, bgy2) - jnp.minimum(py1, bgy1)
        ew = jnp.maximum(ew, 0.0)
        eh = jnp.maximum(eh, 0.0)
        earea = ew * eh
        giou = iou - (earea - union) / (earea + 1e-8)

        cost = 5.0 * bbox + (-prob) + 2.0 * (-giou)

        qidx = base_iota + (qi * qb + i * _TILE)
        am, ai = accs[i % 4]
        lt = cost < am
        accs[i % 4] = [jnp.where(lt, cost, am), jnp.where(lt, qidx, ai)]

    blk_min = jnp.minimum(jnp.minimum(accs[0][0], accs[1][0]),
                          jnp.minimum(accs[2][0], accs[3][0]))
    blk_min = jnp.min(blk_min, axis=0, keepdims=True)
    cand = jnp.full((1, g), _BIG, jnp.int32)
    for am, ai in accs:
        masked = jnp.where(am == blk_min, ai, _BIG)
        cand = jnp.minimum(cand, jnp.min(masked, axis=0, keepdims=True))

    @pl.when(qi == 0)
    def _init():
        min_ref[0] = jnp.full_like(min_ref[0], jnp.inf)
        idx_ref[0] = jnp.zeros_like(idx_ref[0])

    acc_min = min_ref[0]
    better = blk_min < acc_min
    min_ref[0] = jnp.where(better, blk_min, acc_min)
    idx_ref[0] = jnp.where(better, cand, idx_ref[0])


@functools.partial(jax.jit, static_argnames=("qb", "q_limit", "interpret"))
def _match_tc(pred_logits, pred_boxes, gt_boxes, gt_labels, qb=2000,
              q_limit=None, interpret=False):
    B, Q, C = pred_logits.shape
    G = gt_labels.shape[1]
    nq = (Q if q_limit is None else q_limit) // qb

    onehot = (gt_labels[:, None, :] ==
              jnp.arange(C, dtype=gt_labels.dtype)[None, :, None]
              ).astype(jnp.float32)
    gcx, gcy, gw, gh = (gt_boxes[..., 0], gt_boxes[..., 1],
                        gt_boxes[..., 2], gt_boxes[..., 3])
    gx1 = gcx - 0.5 * gw
    gy1 = gcy - 0.5 * gh
    gx2 = gcx + 0.5 * gw
    gy2 = gcy + 0.5 * gh
    garea = (gx2 - gx1) * (gy2 - gy1)
    gtrep = jnp.stack([gcx, gcy, gw, gh, gx1, gy1, gx2, gy2, garea],
                      axis=1)
    gtrep = jnp.repeat(gtrep, _TILE, axis=1)

    kern = functools.partial(_match_block_kernel, qb=qb, g=G)
    min_c, idx = pl.pallas_call(
        kern,
        grid=(B, nq),
        in_specs=[
            pl.BlockSpec((1, C, G), lambda b, qi: (b, 0, 0)),
            pl.BlockSpec((1, 72, G), lambda b, qi: (b, 0, 0)),
            pl.BlockSpec((1, qb, C), lambda b, qi: (b, qi, 0)),
            pl.BlockSpec((1, qb, 4), lambda b, qi: (b, qi, 0)),
        ],
        out_specs=[
            pl.BlockSpec((1, 1, G), lambda b, qi: (b, 0, 0)),
            pl.BlockSpec((1, 1, G), lambda b, qi: (b, 0, 0)),
        ],
        out_shape=[
            jax.ShapeDtypeStruct((B, 1, G), jnp.float32),
            jax.ShapeDtypeStruct((B, 1, G), jnp.int32),
        ],
        interpret=interpret,
    )(onehot, gtrep, pred_logits, pred_boxes)
    return min_c[:, 0, :], idx[:, 0, :]


def _make_sc_matcher(B, Q, C, G, q0, qpw):
    nv = qpw // 16
    owords = B * G * 16
    mesh = plsc.VectorSubcoreMesh(core_axis_name="c", subcore_axis_name="s")

    @functools.partial(
        pl.kernel,
        out_type=[jax.ShapeDtypeStruct((_NW * owords,), jnp.float32),
                  jax.ShapeDtypeStruct((_NW * owords,), jnp.int32)],
        mesh=mesh,
        scratch_types=[
            pltpu.VMEM((C * qpw,), jnp.float32),
            pltpu.VMEM((4 * qpw,), jnp.float32),
            pltpu.VMEM((C * qpw,), jnp.float32),
            pltpu.VMEM((qpw,), jnp.float32),
            pltpu.VMEM((9 * qpw,), jnp.float32),
            pltpu.VMEM((B * G * 16,), jnp.float32),
            pltpu.VMEM((owords,), jnp.float32),
            pltpu.VMEM((owords,), jnp.int32),
        ],
    )
    def sc_kernel(logits_hbm, boxes_hbm, gtrec_hbm,
                  omin_hbm, oidx_hbm,
                  lbuf, bxbuf, ebuf, rsbuf, pbuf, gtbuf,
                  obmin, obidx):
        wid = lax.axis_index("s") * 2 + lax.axis_index("c")
        qstart = q0 + wid * qpw
        pltpu.sync_copy(gtrec_hbm, gtbuf)
        iota = lax.iota(jnp.int32, 16)

        for b in range(B):
            pltpu.sync_copy(
                logits_hbm.at[pl.ds((b * _NW + wid) * C * qpw, C * qpw)],
                lbuf)
            pltpu.sync_copy(
                boxes_hbm.at[pl.ds((b * _NW + wid) * 4 * qpw, 4 * qpw)],
                bxbuf)

            for j in range(nv):
                def mx_body(c, mx):
                    return jnp.maximum(mx, lbuf[pl.ds(c * qpw + j * 16, 16)])
                mx = lax.fori_loop(0, C, mx_body,
                                   jnp.full((16,), -jnp.inf, jnp.float32),
                                   unroll=9)

                def es_body(c, sacc):
                    v = lbuf[pl.ds(c * qpw + j * 16, 16)]
                    e = jnp.exp(v - mx)
                    ebuf[pl.ds(c * qpw + j * 16, 16)] = e
                    return sacc + e
                s = lax.fori_loop(0, C, es_body,
                                  jnp.zeros((16,), jnp.float32), unroll=9)
                rsbuf[pl.ds(j * 16, 16)] = 1.0 / s

                pcx = bxbuf[pl.ds(0 * qpw + j * 16, 16)]
                pcy = bxbuf[pl.ds(1 * qpw + j * 16, 16)]
                pw = bxbuf[pl.ds(2 * qpw + j * 16, 16)]
                ph = bxbuf[pl.ds(3 * qpw + j * 16, 16)]
                px1 = pcx - 0.5 * pw
                py1 = pcy - 0.5 * ph
                px2 = pcx + 0.5 * pw
                py2 = pcy + 0.5 * ph
                pbuf[pl.ds(0 * qpw + j * 16, 16)] = px1
                pbuf[pl.ds(1 * qpw + j * 16, 16)] = py1
                pbuf[pl.ds(2 * qpw + j * 16, 16)] = px2
                pbuf[pl.ds(3 * qpw + j * 16, 16)] = py2
                pbuf[pl.ds(4 * qpw + j * 16, 16)] = (px2 - px1) * (py2 - py1)

            def g_body(g, carry):
                rec = gtbuf[pl.ds((b * G + g) * 16, 16)]
                gcx, gcy, gw, gh = rec[0], rec[1], rec[2], rec[3]
                gx1, gy1, gx2, gy2 = rec[4], rec[5], rec[6], rec[7]
                garea = rec[8]
                eoff = rec[9].astype(jnp.int32) * qpw

                bmin = jnp.full((16,), jnp.inf, jnp.float32)
                bidx = jnp.zeros((16,), jnp.int32)
                for j in range(nv):
                    pcx = bxbuf[pl.ds(0 * qpw + j * 16, 16)]
                    pcy = bxbuf[pl.ds(1 * qpw + j * 16, 16)]
                    pw = bxbuf[pl.ds(2 * qpw + j * 16, 16)]
                    ph = bxbuf[pl.ds(3 * qpw + j * 16, 16)]
                    px1 = pbuf[pl.ds(0 * qpw + j * 16, 16)]
                    py1 = pbuf[pl.ds(1 * qpw + j * 16, 16)]
                    px2 = pbuf[pl.ds(2 * qpw + j * 16, 16)]
                    py2 = pbuf[pl.ds(3 * qpw + j * 16, 16)]
                    parea = pbuf[pl.ds(4 * qpw + j * 16, 16)]
                    el = ebuf[pl.ds(eoff + j * 16, 16)]
                    rs = rsbuf[pl.ds(j * 16, 16)]
                    prob = el * rs

                    bbox = (jnp.abs(pcx - gcx) + jnp.abs(pcy - gcy)
                            + jnp.abs(pw - gw) + jnp.abs(ph - gh))
                    iw = jnp.maximum(jnp.minimum(px2, gx2)
                                     - jnp.maximum(px1, gx1), 0.0)
                    ih = jnp.maximum(jnp.minimum(py2, gy2)
                                     - jnp.maximum(py1, gy1), 0.0)
                    inter = iw * ih
                    union = parea + garea - inter
                    iou = inter / (union + 1e-8)
                    ew = jnp.maximum(jnp.maximum(px2, gx2)
                                     - jnp.minimum(px1, gx1), 0.0)
                    eh = jnp.maximum(jnp.maximum(py2, gy2)
                                     - jnp.minimum(py1, gy1), 0.0)
                    earea = ew * eh
                    giou = iou - (earea - union) / (earea + 1e-8)
                    cost = 5.0 * bbox + (-prob) + 2.0 * (-giou)

                    qidx = iota + (qstart + j * 16)
                    ltm = cost < bmin
                    bmin = jnp.where(ltm, cost, bmin)
                    bidx = jnp.where(ltm, qidx, bidx)

                obmin[pl.ds((b * G + g) * 16, 16)] = bmin
                obidx[pl.ds((b * G + g) * 16, 16)] = bidx
                return carry
            lax.fori_loop(0, G, g_body, 0)

        pltpu.sync_copy(obmin, omin_hbm.at[pl.ds(wid * owords, owords)])
        pltpu.sync_copy(obidx, oidx_hbm.at[pl.ds(wid * owords, owords)])

    return sc_kernel


def _sc_layout_kernel(l_ref, bx_ref, lt_ref, bxt_ref):
    lt_ref[0, 0] = l_ref[0].T
    bxt_ref[0, 0] = bx_ref[0].T


def _sc_layout(logits_sl, boxes_sl, qpw):
    B, qsc, C = logits_sl.shape
    nw = qsc // qpw
    lt, bxt = pl.pallas_call(
        _sc_layout_kernel,
        grid=(B, nw),
        in_specs=[pl.BlockSpec((1, qpw, C), lambda b, w: (b, w, 0)),
                  pl.BlockSpec((1, qpw, 4), lambda b, w: (b, w, 0))],
        out_specs=[pl.BlockSpec((1, 1, C, qpw), lambda b, w: (b, w, 0, 0)),
                   pl.BlockSpec((1, 1, 4, qpw), lambda b, w: (b, w, 0, 0))],
        out_shape=[jax.ShapeDtypeStruct((B, nw, C, qpw), jnp.float32),
                   jax.ShapeDtypeStruct((B, nw, 4, qpw), jnp.float32)],
    )(logits_sl, boxes_sl)
    return lt.reshape(-1), bxt.reshape(-1)


def kernel(pred_logits, pred_boxes, gt_boxes, gt_labels):
    B, Q, C = pred_logits.shape
    G = gt_labels.shape[1]
    q0 = Q - _NW * _QPW
    qtc = -(-q0 // _QB_TC) * _QB_TC

    gcx, gcy, gw, gh = (gt_boxes[..., 0], gt_boxes[..., 1],
                        gt_boxes[..., 2], gt_boxes[..., 3])
    gx1 = gcx - 0.5 * gw
    gy1 = gcy - 0.5 * gh
    gx2 = gcx + 0.5 * gw
    gy2 = gcy + 0.5 * gh
    garea = (gx2 - gx1) * (gy2 - gy1)
    labf = gt_labels.astype(jnp.float32)
    zero = jnp.zeros_like(gcx)
    gtrec = jnp.stack([gcx, gcy, gw, gh, gx1, gy1, gx2, gy2, garea, labf,
                       zero, zero, zero, zero, zero, zero],
                      axis=2).reshape(-1)

    lt, bxt = _sc_layout(pred_logits[:, q0:, :], pred_boxes[:, q0:, :], _QPW)

    sc = _make_sc_matcher(B, Q, C, G, q0, _QPW)
    sc_min, sc_idx = sc(lt, bxt, gtrec)

    tc_min, tc_idx = _match_tc(pred_logits, pred_boxes, gt_boxes, gt_labels,
                               qb=_QB_TC, q_limit=qtc)

    sc_min = sc_min.reshape(_NW, B, G, 16)
    sc_idx = sc_idx.reshape(_NW, B, G, 16)

    sgmin = jnp.min(sc_min, axis=(0, 3))
    sgidx = jnp.min(jnp.where(sc_min == sgmin[None, :, :, None],
                              sc_idx, _BIG), axis=(0, 3))

    all_min = jnp.stack([tc_min, sgmin], axis=0)
    all_idx = jnp.stack([tc_idx, sgidx], axis=0)
    gmin = jnp.min(all_min, axis=0)
    gidx = jnp.min(jnp.where(all_min == gmin[None], all_idx, _BIG), axis=0)
    return gmin, gidx.astype(jnp.int32)


# speedup vs baseline: 1.5892x; 1.5892x over previous
import functools

import jax
import jax.numpy as jnp
from jax import lax
from jax.experimental import pallas as pl
from jax.experimental.pallas import tpu as pltpu
from jax.experimental.pallas import tpu_sc as plsc

_BIG = 2**30
_NW = 32
_QPW = 128
_QB_TC = 1024
_GP = 104


def _match_block_kernel(onehotT_ref, gtl_ref, logitsT_ref, pboxesT_ref,
                        min_ref, idx_ref, *, qb: int, nb: int):
    qi = pl.program_id(0)
    nqt = qb // 128
    ngt = _GP // 8
    laneiota = jax.lax.broadcasted_iota(jnp.int32, (8, 128), 1)

    @pl.when(qi == 0)
    def _init():
        min_ref[...] = jnp.full_like(min_ref[...], jnp.inf)
        idx_ref[...] = jnp.zeros_like(idx_ref[...])

    for b in range(nb):
        l = logitsT_ref[:, b, :]
        m = jnp.max(l, axis=0, keepdims=True)
        e = jnp.exp(l - m)
        s = jnp.sum(e, axis=0, keepdims=True)
        eg = jnp.dot(onehotT_ref[b], e,
                     preferred_element_type=jnp.float32,
                     precision=jax.lax.Precision.HIGHEST)

        pbv = pboxesT_ref[b]
        pcx, pcy, pw, ph = (pbv[0:1], pbv[1:2], pbv[2:3], pbv[3:4])
        px1 = pcx - 0.5 * pw
        py1 = pcy - 0.5 * ph
        px2 = pcx + 0.5 * pw
        py2 = pcy + 0.5 * ph
        parea = (px2 - px1) * (py2 - py1)

        gtv = gtl_ref[b]

        accs = [[jnp.full((8, 128), jnp.inf, jnp.float32),
                 jnp.zeros((8, 128), jnp.int32)] for _ in range(ngt)]

        for qt in range(nqt):
            qsl = slice(qt * 128, (qt + 1) * 128)
            bcx = jnp.broadcast_to(pcx[:, qsl], (8, 128))
            bcy = jnp.broadcast_to(pcy[:, qsl], (8, 128))
            bw = jnp.broadcast_to(pw[:, qsl], (8, 128))
            bh = jnp.broadcast_to(ph[:, qsl], (8, 128))
            bx1 = jnp.broadcast_to(px1[:, qsl], (8, 128))
            by1 = jnp.broadcast_to(py1[:, qsl], (8, 128))
            bx2 = jnp.broadcast_to(px2[:, qsl], (8, 128))
            by2 = jnp.broadcast_to(py2[:, qsl], (8, 128))
            bparea = jnp.broadcast_to(parea[:, qsl], (8, 128))
            bs = jnp.broadcast_to(s[:, qsl], (8, 128))
            qvec = laneiota + (qi * qb + qt * 128)

            for gi in range(ngt):
                r = gi * 8
                ggcx = gtv[0 * _GP + r:0 * _GP + r + 8, :]
                ggcy = gtv[1 * _GP + r:1 * _GP + r + 8, :]
                ggw = gtv[2 * _GP + r:2 * _GP + r + 8, :]
                ggh = gtv[3 * _GP + r:3 * _GP + r + 8, :]
                ggx1 = gtv[4 * _GP + r:4 * _GP + r + 8, :]
                ggy1 = gtv[5 * _GP + r:5 * _GP + r + 8, :]
                ggx2 = gtv[6 * _GP + r:6 * _GP + r + 8, :]
                ggy2 = gtv[7 * _GP + r:7 * _GP + r + 8, :]
                ggarea = gtv[8 * _GP + r:8 * _GP + r + 8, :]

                prob = eg[r:r + 8, qsl] / bs

                bbox = (jnp.abs(bcx - ggcx) + jnp.abs(bcy - ggcy)
                        + jnp.abs(bw - ggw) + jnp.abs(bh - ggh))
                iw = jnp.maximum(jnp.minimum(bx2, ggx2)
                                 - jnp.maximum(bx1, ggx1), 0.0)
                ih = jnp.maximum(jnp.minimum(by2, ggy2)
                                 - jnp.maximum(by1, ggy1), 0.0)
                inter = iw * ih
                union = bparea + ggarea - inter
                iou = inter / (union + 1e-8)
                ew = jnp.maximum(jnp.maximum(bx2, ggx2)
                                 - jnp.minimum(bx1, ggx1), 0.0)
                eh = jnp.maximum(jnp.maximum(by2, ggy2)
                                 - jnp.minimum(by1, ggy1), 0.0)
                earea = ew * eh
                giou = iou - (earea - union) / (earea + 1e-8)
                cost = 5.0 * bbox + (-prob) + 2.0 * (-giou)

                am, ai = accs[gi]
                ltm = cost < am
                accs[gi] = [jnp.where(ltm, cost, am),
                            jnp.where(ltm, qvec, ai)]

        mins, idxs = [], []
        for gi in range(ngt):
            am, ai = accs[gi]
            m8 = jnp.min(am, axis=1, keepdims=True)
            i8 = jnp.min(jnp.where(am == m8, ai, _BIG),
                         axis=1, keepdims=True)
            mins.append(m8)
            idxs.append(i8)
        blk_min = jnp.concatenate(mins, axis=0)
        blk_idx = jnp.concatenate(idxs, axis=0)

        acc_min = min_ref[b]
        better = blk_min < acc_min
        min_ref[b] = jnp.where(better, blk_min, acc_min)
        idx_ref[b] = jnp.where(better, blk_idx, idx_ref[b])


@functools.partial(jax.jit, static_argnames=("qb", "q_limit"))
def _match_tc(logitsT, pboxesT, gt_boxes, gt_labels, qb=1024, q_limit=None):
    C, B, Q = logitsT.shape
    G = gt_labels.shape[1]
    nq = (Q if q_limit is None else q_limit) // qb

    onehotT = (gt_labels[:, :, None] ==
               jnp.arange(C, dtype=gt_labels.dtype)[None, None, :]
               ).astype(jnp.float32)
    onehotT = jnp.concatenate(
        [onehotT, jnp.zeros((B, _GP - G, C), jnp.float32)], axis=1)

    gcx, gcy, gw, gh = (gt_boxes[..., 0], gt_boxes[..., 1],
                        gt_boxes[..., 2], gt_boxes[..., 3])
    gx1 = gcx - 0.5 * gw
    gy1 = gcy - 0.5 * gh
    gx2 = gcx + 0.5 * gw
    gy2 = gcy + 0.5 * gh
    garea = (gx2 - gx1) * (gy2 - gy1)
    gvals = jnp.stack([gcx, gcy, gw, gh, gx1, gy1, gx2, gy2, garea],
                      axis=1)
    gvals = jnp.concatenate(
        [gvals, jnp.zeros((B, 9, _GP - G), jnp.float32)], axis=2)
    gtl = jnp.broadcast_to(gvals[..., None], (B, 9, _GP, 128)
                           ).reshape(B, 9 * _GP, 128)

    kern = functools.partial(_match_block_kernel, qb=qb, nb=B)
    min_c, idx = pl.pallas_call(
        kern,
        grid=(nq,),
        in_specs=[
            pl.BlockSpec((B, _GP, C), lambda qi: (0, 0, 0)),
            pl.BlockSpec((B, 9 * _GP, 128), lambda qi: (0, 0, 0)),
            pl.BlockSpec((C, B, qb), lambda qi: (0, 0, qi)),
            pl.BlockSpec((B, 4, qb), lambda qi: (0, 0, qi)),
        ],
        out_specs=[
            pl.BlockSpec((B, _GP, 1), lambda qi: (0, 0, 0)),
            pl.BlockSpec((B, _GP, 1), lambda qi: (0, 0, 0)),
        ],
        out_shape=[
            jax.ShapeDtypeStruct((B, _GP, 1), jnp.float32),
            jax.ShapeDtypeStruct((B, _GP, 1), jnp.int32),
        ],
    )(onehotT, gtl, logitsT, pboxesT)
    return min_c[:, :G, 0], idx[:, :G, 0]


def _make_sc_matcher(B, Q, C, G, q0, qpw):
    nv = qpw // 16
    owords = B * G * 16
    mesh = plsc.VectorSubcoreMesh(core_axis_name="c", subcore_axis_name="s")

    @functools.partial(
        pl.kernel,
        out_type=[jax.ShapeDtypeStruct((_NW * owords,), jnp.float32),
                  jax.ShapeDtypeStruct((_NW * owords,), jnp.int32)],
        mesh=mesh,
        scratch_types=[
            pltpu.VMEM((C * qpw,), jnp.float32),
            pltpu.VMEM((4 * qpw,), jnp.float32),
            pltpu.VMEM((C * qpw,), jnp.float32),
            pltpu.VMEM((qpw,), jnp.float32),
            pltpu.VMEM((5 * qpw,), jnp.float32),
            pltpu.VMEM((B * G * 16,), jnp.float32),
            pltpu.VMEM((owords,), jnp.float32),
            pltpu.VMEM((owords,), jnp.int32),
            pltpu.SemaphoreType.DMA,
        ],
    )
    def sc_kernel(logits_hbm, boxes_hbm, gtrec_hbm,
                  omin_hbm, oidx_hbm,
                  lbuf, bxbuf, ebuf, rsbuf, pbuf, gtbuf,
                  obmin, obidx, sem):
        wid = lax.axis_index("s") * 2 + lax.axis_index("c")
        qstart = q0 + wid * qpw
        pltpu.sync_copy(gtrec_hbm, gtbuf)
        iota = lax.iota(jnp.int32, 16)

        for b in range(B):
            def dma_body(c, carry):
                pltpu.async_copy(
                    logits_hbm.at[pl.ds((c * B + b) * Q + qstart, qpw)],
                    lbuf.at[pl.ds(c * qpw, qpw)], sem)
                return carry
            lax.fori_loop(0, C, dma_body, 0)
            for k in range(4):
                pltpu.async_copy(
                    boxes_hbm.at[pl.ds((b * 4 + k) * Q + qstart, qpw)],
                    bxbuf.at[pl.ds(k * qpw, qpw)], sem)
            pltpu.make_async_copy(logits_hbm.at[pl.ds(0, C * qpw)],
                                  lbuf, sem).wait()
            pltpu.make_async_copy(boxes_hbm.at[pl.ds(0, 4 * qpw)],
                                  bxbuf, sem).wait()

            def stats_body(j, carry):
                j16 = j * 16

                def mx_body(c, mx):
                    return jnp.maximum(mx, lbuf[pl.ds(c * qpw + j16, 16)])
                mx = lax.fori_loop(0, C, mx_body,
                                   jnp.full((16,), -jnp.inf, jnp.float32),
                                   unroll=9)

                def es_body(c, sacc):
                    v = lbuf[pl.ds(c * qpw + j16, 16)]
                    e = jnp.exp(v - mx)
                    ebuf[pl.ds(c * qpw + j16, 16)] = e
                    return sacc + e
                s = lax.fori_loop(0, C, es_body,
                                  jnp.zeros((16,), jnp.float32), unroll=9)
                rsbuf[pl.ds(j16, 16)] = 1.0 / s

                pcx = bxbuf[pl.ds(0 * qpw + j16, 16)]
                pcy = bxbuf[pl.ds(1 * qpw + j16, 16)]
                pw = bxbuf[pl.ds(2 * qpw + j16, 16)]
                ph = bxbuf[pl.ds(3 * qpw + j16, 16)]
                px1 = pcx - 0.5 * pw
                py1 = pcy - 0.5 * ph
                px2 = pcx + 0.5 * pw
                py2 = pcy + 0.5 * ph
                pbuf[pl.ds(0 * qpw + j16, 16)] = px1
                pbuf[pl.ds(1 * qpw + j16, 16)] = py1
                pbuf[pl.ds(2 * qpw + j16, 16)] = px2
                pbuf[pl.ds(3 * qpw + j16, 16)] = py2
                pbuf[pl.ds(4 * qpw + j16, 16)] = (px2 - px1) * (py2 - py1)
                return carry
            lax.fori_loop(0, nv, stats_body, 0)

            def g_body(g, carry):
                rec = gtbuf[pl.ds((b * G + g) * 16, 16)]
                gcx, gcy, gw, gh = rec[0], rec[1], rec[2], rec[3]
                gx1, gy1, gx2, gy2 = rec[4], rec[5], rec[6], rec[7]
                garea = rec[8]
                eoff = rec[9].astype(jnp.int32) * qpw

                bmin = jnp.full((16,), jnp.inf, jnp.float32)
                bidx = jnp.zeros((16,), jnp.int32)
                for j in range(nv):
                    pcx = bxbuf[pl.ds(0 * qpw + j * 16, 16)]
                    pcy = bxbuf[pl.ds(1 * qpw + j * 16, 16)]
                    pw = bxbuf[pl.ds(2 * qpw + j * 16, 16)]
                    ph = bxbuf[pl.ds(3 * qpw + j * 16, 16)]
                    px1 = pbuf[pl.ds(0 * qpw + j * 16, 16)]
                    py1 = pbuf[pl.ds(1 * qpw + j * 16, 16)]
                    px2 = pbuf[pl.ds(2 * qpw + j * 16, 16)]
                    py2 = pbuf[pl.ds(3 * qpw + j * 16, 16)]
                    parea = pbuf[pl.ds(4 * qpw + j * 16, 16)]
                    el = ebuf[pl.ds(eoff + j * 16, 16)]
                    rs = rsbuf[pl.ds(j * 16, 16)]
                    prob = el * rs

                    bbox = (jnp.abs(pcx - gcx) + jnp.abs(pcy - gcy)
                            + jnp.abs(pw - gw) + jnp.abs(ph - gh))
                    iw = jnp.maximum(jnp.minimum(px2, gx2)
                                     - jnp.maximum(px1, gx1), 0.0)
                    ih = jnp.maximum(jnp.minimum(py2, gy2)
                                     - jnp.maximum(py1, gy1), 0.0)
                    inter = iw * ih
                    union = parea + garea - inter
                    iou = inter / (union + 1e-8)
                    ew = jnp.maximum(jnp.maximum(px2, gx2)
                                     - jnp.minimum(px1, gx1), 0.0)
                    eh = jnp.maximum(jnp.maximum(py2, gy2)
                                     - jnp.minimum(py1, gy1), 0.0)
                    earea = ew * eh
                    giou = iou - (earea - union) / (earea + 1e-8)
                    cost = 5.0 * bbox + (-prob) + 2.0 * (-giou)

                    qidx = iota + (qstart + j * 16)
                    ltm = cost < bmin
                    bmin = jnp.where(ltm, cost, bmin)
                    bidx = jnp.where(ltm, qidx, bidx)

                obmin[pl.ds((b * G + g) * 16, 16)] = bmin
                obidx[pl.ds((b * G + g) * 16, 16)] = bidx
                return carry
            lax.fori_loop(0, G, g_body, 0)

        pltpu.sync_copy(obmin, omin_hbm.at[pl.ds(wid * owords, owords)])
        pltpu.sync_copy(obidx, oidx_hbm.at[pl.ds(wid * owords, owords)])

    return sc_kernel


def kernel(pred_logits, pred_boxes, gt_boxes, gt_labels):
    B, Q, C = pred_logits.shape
    G = gt_labels.shape[1]
    q0 = Q - _NW * _QPW
    qtc = -(-q0 // _QB_TC) * _QB_TC

    logitsT = jnp.transpose(pred_logits, (2, 0, 1))
    pboxesT = jnp.transpose(pred_boxes, (0, 2, 1))

    gcx, gcy, gw, gh = (gt_boxes[..., 0], gt_boxes[..., 1],
                        gt_boxes[..., 2], gt_boxes[..., 3])
    gx1 = gcx - 0.5 * gw
    gy1 = gcy - 0.5 * gh
    gx2 = gcx + 0.5 * gw
    gy2 = gcy + 0.5 * gh
    garea = (gx2 - gx1) * (gy2 - gy1)
    labf = gt_labels.astype(jnp.float32)
    zero = jnp.zeros_like(gcx)
    gtrec = jnp.stack([gcx, gcy, gw, gh, gx1, gy1, gx2, gy2, garea, labf,
                       zero, zero, zero, zero, zero, zero],
                      axis=2).reshape(-1)

    sc = _make_sc_matcher(B, Q, C, G, q0, _QPW)
    sc_min, sc_idx = sc(logitsT.reshape(-1), pboxesT.reshape(-1), gtrec)

    tc_min, tc_idx = _match_tc(logitsT, pboxesT, gt_boxes, gt_labels,
                               qb=_QB_TC, q_limit=qtc)

    sc_min = sc_min.reshape(_NW, B, G, 16)
    sc_idx = sc_idx.reshape(_NW, B, G, 16)
    sgmin = jnp.min(sc_min, axis=(0, 3))
    sgidx = jnp.min(jnp.where(sc_min == sgmin[None, :, :, None],
                              sc_idx, _BIG), axis=(0, 3))

    all_min = jnp.stack([tc_min, sgmin], axis=0)
    all_idx = jnp.stack([tc_idx, sgidx], axis=0)
    gmin = jnp.min(all_min, axis=0)
    gidx = jnp.min(jnp.where(all_min == gmin[None], all_idx, _BIG), axis=0)
    return gmin, gidx.astype(jnp.int32)


# speedup vs baseline: 2.4676x; 1.5527x over previous
import functools

import jax
import jax.numpy as jnp
from jax import lax
from jax.experimental import pallas as pl
from jax.experimental.pallas import tpu as pltpu
from jax.experimental.pallas import tpu_sc as plsc

_BIG = 2**30
_NW = 32
_QPW = 128
_QB_TC = 1024
_GP = 104


def _match_block_kernel(onehotT_ref, gtl_ref, logitsT_ref, pboxesT_ref,
                        min_ref, idx_ref, *, qb: int, nb: int):
    qi = pl.program_id(0)
    nqt = qb // 128
    ngt = _GP // 8
    laneiota = jax.lax.broadcasted_iota(jnp.int32, (8, 128), 1)

    @pl.when(qi == 0)
    def _init():
        min_ref[...] = jnp.full_like(min_ref[...], jnp.inf)
        idx_ref[...] = jnp.zeros_like(idx_ref[...])

    for b in range(nb):
        l = logitsT_ref[:, b, :]
        m = jnp.max(l, axis=0, keepdims=True)
        e = jnp.exp(l - m)
        s = jnp.sum(e, axis=0, keepdims=True)
        eg = jnp.dot(onehotT_ref[b], e,
                     preferred_element_type=jnp.float32,
                     precision=jax.lax.Precision.HIGHEST)

        pbv = pboxesT_ref[b]
        pcx, pcy, pw, ph = (pbv[0:1], pbv[1:2], pbv[2:3], pbv[3:4])
        px1 = pcx - 0.5 * pw
        py1 = pcy - 0.5 * ph
        px2 = pcx + 0.5 * pw
        py2 = pcy + 0.5 * ph
        parea = (px2 - px1) * (py2 - py1)

        gtv = gtl_ref[b]

        accs = [[jnp.full((8, 128), jnp.inf, jnp.float32),
                 jnp.zeros((8, 128), jnp.int32)] for _ in range(ngt)]

        for qt in range(nqt):
            qsl = slice(qt * 128, (qt + 1) * 128)
            bcx = jnp.broadcast_to(pcx[:, qsl], (8, 128))
            bcy = jnp.broadcast_to(pcy[:, qsl], (8, 128))
            bw = jnp.broadcast_to(pw[:, qsl], (8, 128))
            bh = jnp.broadcast_to(ph[:, qsl], (8, 128))
            bx1 = jnp.broadcast_to(px1[:, qsl], (8, 128))
            by1 = jnp.broadcast_to(py1[:, qsl], (8, 128))
            bx2 = jnp.broadcast_to(px2[:, qsl], (8, 128))
            by2 = jnp.broadcast_to(py2[:, qsl], (8, 128))
            bparea = jnp.broadcast_to(parea[:, qsl], (8, 128))
            bs = jnp.broadcast_to(s[:, qsl], (8, 128))
            qvec = laneiota + (qi * qb + qt * 128)

            for gi in range(ngt):
                r = gi * 8
                ggcx = gtv[0 * _GP + r:0 * _GP + r + 8, :]
                ggcy = gtv[1 * _GP + r:1 * _GP + r + 8, :]
                ggw = gtv[2 * _GP + r:2 * _GP + r + 8, :]
                ggh = gtv[3 * _GP + r:3 * _GP + r + 8, :]
                ggx1 = gtv[4 * _GP + r:4 * _GP + r + 8, :]
                ggy1 = gtv[5 * _GP + r:5 * _GP + r + 8, :]
                ggx2 = gtv[6 * _GP + r:6 * _GP + r + 8, :]
                ggy2 = gtv[7 * _GP + r:7 * _GP + r + 8, :]
                ggarea = gtv[8 * _GP + r:8 * _GP + r + 8, :]

                prob = eg[r:r + 8, qsl] / bs

                bbox = (jnp.abs(bcx - ggcx) + jnp.abs(bcy - ggcy)
                        + jnp.abs(bw - ggw) + jnp.abs(bh - ggh))
                iw = jnp.maximum(jnp.minimum(bx2, ggx2)
                                 - jnp.maximum(bx1, ggx1), 0.0)
                ih = jnp.maximum(jnp.minimum(by2, ggy2)
                                 - jnp.maximum(by1, ggy1), 0.0)
                inter = iw * ih
                union = bparea + ggarea - inter
                iou = inter / (union + 1e-8)
                ew = jnp.maximum(jnp.maximum(bx2, ggx2)
                                 - jnp.minimum(bx1, ggx1), 0.0)
                eh = jnp.maximum(jnp.maximum(by2, ggy2)
                                 - jnp.minimum(by1, ggy1), 0.0)
                earea = ew * eh
                giou = iou - (earea - union) / (earea + 1e-8)
                cost = 5.0 * bbox + (-prob) + 2.0 * (-giou)

                am, ai = accs[gi]
                ltm = cost < am
                accs[gi] = [jnp.where(ltm, cost, am),
                            jnp.where(ltm, qvec, ai)]

        mins, idxs = [], []
        for gi in range(ngt):
            am, ai = accs[gi]
            m8 = jnp.min(am, axis=1, keepdims=True)
            i8 = jnp.min(jnp.where(am == m8, ai, _BIG),
                         axis=1, keepdims=True)
            mins.append(m8)
            idxs.append(i8)
        blk_min = jnp.concatenate(mins, axis=0)
        blk_idx = jnp.concatenate(idxs, axis=0)

        acc_min = min_ref[b]
        better = blk_min < acc_min
        min_ref[b] = jnp.where(better, blk_min, acc_min)
        idx_ref[b] = jnp.where(better, blk_idx, idx_ref[b])


@functools.partial(jax.jit, static_argnames=("qb", "q_limit"))
def _match_tc(logitsT, pboxesT, gt_boxes, gt_labels, qb=1024, q_limit=None):
    C, B, Q = logitsT.shape
    G = gt_labels.shape[1]
    nq = (Q if q_limit is None else q_limit) // qb

    onehotT = (gt_labels[:, :, None] ==
               jnp.arange(C, dtype=gt_labels.dtype)[None, None, :]
               ).astype(jnp.float32)
    onehotT = jnp.concatenate(
        [onehotT, jnp.zeros((B, _GP - G, C), jnp.float32)], axis=1)

    gcx, gcy, gw, gh = (gt_boxes[..., 0], gt_boxes[..., 1],
                        gt_boxes[..., 2], gt_boxes[..., 3])
    gx1 = gcx - 0.5 * gw
    gy1 = gcy - 0.5 * gh
    gx2 = gcx + 0.5 * gw
    gy2 = gcy + 0.5 * gh
    garea = (gx2 - gx1) * (gy2 - gy1)
    gvals = jnp.stack([gcx, gcy, gw, gh, gx1, gy1, gx2, gy2, garea],
                      axis=1)
    gvals = jnp.concatenate(
        [gvals, jnp.zeros((B, 9, _GP - G), jnp.float32)], axis=2)
    gtl = jnp.broadcast_to(gvals[..., None], (B, 9, _GP, 128)
                           ).reshape(B, 9 * _GP, 128)

    kern = functools.partial(_match_block_kernel, qb=qb, nb=B)
    min_c, idx = pl.pallas_call(
        kern,
        grid=(nq,),
        in_specs=[
            pl.BlockSpec((B, _GP, C), lambda qi: (0, 0, 0)),
            pl.BlockSpec((B, 9 * _GP, 128), lambda qi: (0, 0, 0)),
            pl.BlockSpec((C, B, qb), lambda qi: (0, 0, qi)),
            pl.BlockSpec((B, 4, qb), lambda qi: (0, 0, qi)),
        ],
        out_specs=[
            pl.BlockSpec((B, _GP, 1), lambda qi: (0, 0, 0)),
            pl.BlockSpec((B, _GP, 1), lambda qi: (0, 0, 0)),
        ],
        out_shape=[
            jax.ShapeDtypeStruct((B, _GP, 1), jnp.float32),
            jax.ShapeDtypeStruct((B, _GP, 1), jnp.int32),
        ],
    )(onehotT, gtl, logitsT, pboxesT)
    return min_c[:, :G, 0], idx[:, :G, 0]


def _make_sc_matcher(B, Q, C, G, q0, qpw):
    nv = qpw // 16
    owords = B * G * 16
    mesh = plsc.VectorSubcoreMesh(core_axis_name="c", subcore_axis_name="s")

    @functools.partial(
        pl.kernel,
        out_type=[jax.ShapeDtypeStruct((_NW * owords,), jnp.float32),
                  jax.ShapeDtypeStruct((_NW * owords,), jnp.int32)],
        mesh=mesh,
        scratch_types=[
            pltpu.VMEM((C * qpw,), jnp.float32),
            pltpu.VMEM((4 * qpw,), jnp.float32),
            pltpu.VMEM((C * qpw,), jnp.float32),
            pltpu.VMEM((qpw,), jnp.float32),
            pltpu.VMEM((5 * qpw,), jnp.float32),
            pltpu.VMEM((B * G * 16,), jnp.float32),
            pltpu.VMEM((owords,), jnp.float32),
            pltpu.VMEM((owords,), jnp.int32),
            pltpu.SemaphoreType.DMA,
        ],
    )
    def sc_kernel(logits_hbm, boxes_hbm, gtrec_hbm,
                  omin_hbm, oidx_hbm,
                  lbuf, bxbuf, ebuf, rsbuf, pbuf, gtbuf,
                  obmin, obidx, sem):
        wid = lax.axis_index("s") * 2 + lax.axis_index("c")
        qloc = wid * qpw
        qstart = q0 + qloc
        pltpu.sync_copy(gtrec_hbm, gtbuf)
        iota = lax.iota(jnp.int32, 16)

        for b in range(B):
            def dma_body(c, carry):
                pltpu.async_copy(
                    logits_hbm.at[pl.ds((c * B + b) * Q + qloc, qpw)],
                    lbuf.at[pl.ds(c * qpw, qpw)], sem)
                return carry
            lax.fori_loop(0, C, dma_body, 0)
            for k in range(4):
                pltpu.async_copy(
                    boxes_hbm.at[pl.ds((b * 4 + k) * Q + qloc, qpw)],
                    bxbuf.at[pl.ds(k * qpw, qpw)], sem)
            pltpu.make_async_copy(logits_hbm.at[pl.ds(0, C * qpw)],
                                  lbuf, sem).wait()
            pltpu.make_async_copy(boxes_hbm.at[pl.ds(0, 4 * qpw)],
                                  bxbuf, sem).wait()

            def stats_body(j, carry):
                j16 = j * 16

                def mx_body(c, mx):
                    return jnp.maximum(mx, lbuf[pl.ds(c * qpw + j16, 16)])
                mx = lax.fori_loop(0, C, mx_body,
                                   jnp.full((16,), -jnp.inf, jnp.float32),
                                   unroll=9)

                def es_body(c, sacc):
                    v = lbuf[pl.ds(c * qpw + j16, 16)]
                    e = jnp.exp(v - mx)
                    ebuf[pl.ds(c * qpw + j16, 16)] = e
                    return sacc + e
                s = lax.fori_loop(0, C, es_body,
                                  jnp.zeros((16,), jnp.float32), unroll=9)
                rsbuf[pl.ds(j16, 16)] = 1.0 / s

                pcx = bxbuf[pl.ds(0 * qpw + j16, 16)]
                pcy = bxbuf[pl.ds(1 * qpw + j16, 16)]
                pw = bxbuf[pl.ds(2 * qpw + j16, 16)]
                ph = bxbuf[pl.ds(3 * qpw + j16, 16)]
                px1 = pcx - 0.5 * pw
                py1 = pcy - 0.5 * ph
                px2 = pcx + 0.5 * pw
                py2 = pcy + 0.5 * ph
                pbuf[pl.ds(0 * qpw + j16, 16)] = px1
                pbuf[pl.ds(1 * qpw + j16, 16)] = py1
                pbuf[pl.ds(2 * qpw + j16, 16)] = px2
                pbuf[pl.ds(3 * qpw + j16, 16)] = py2
                pbuf[pl.ds(4 * qpw + j16, 16)] = (px2 - px1) * (py2 - py1)
                return carry
            lax.fori_loop(0, nv, stats_body, 0)

            def g_body(g, carry):
                rec = gtbuf[pl.ds((b * G + g) * 16, 16)]
                gcx, gcy, gw, gh = rec[0], rec[1], rec[2], rec[3]
                gx1, gy1, gx2, gy2 = rec[4], rec[5], rec[6], rec[7]
                garea = rec[8]
                eoff = rec[9].astype(jnp.int32) * qpw

                bmin = jnp.full((16,), jnp.inf, jnp.float32)
                bidx = jnp.zeros((16,), jnp.int32)
                for j in range(nv):
                    pcx = bxbuf[pl.ds(0 * qpw + j * 16, 16)]
                    pcy = bxbuf[pl.ds(1 * qpw + j * 16, 16)]
                    pw = bxbuf[pl.ds(2 * qpw + j * 16, 16)]
                    ph = bxbuf[pl.ds(3 * qpw + j * 16, 16)]
                    px1 = pbuf[pl.ds(0 * qpw + j * 16, 16)]
                    py1 = pbuf[pl.ds(1 * qpw + j * 16, 16)]
                    px2 = pbuf[pl.ds(2 * qpw + j * 16, 16)]
                    py2 = pbuf[pl.ds(3 * qpw + j * 16, 16)]
                    parea = pbuf[pl.ds(4 * qpw + j * 16, 16)]
                    el = ebuf[pl.ds(eoff + j * 16, 16)]
                    rs = rsbuf[pl.ds(j * 16, 16)]
                    prob = el * rs

                    bbox = (jnp.abs(pcx - gcx) + jnp.abs(pcy - gcy)
                            + jnp.abs(pw - gw) + jnp.abs(ph - gh))
                    iw = jnp.maximum(jnp.minimum(px2, gx2)
                                     - jnp.maximum(px1, gx1), 0.0)
                    ih = jnp.maximum(jnp.minimum(py2, gy2)
                                     - jnp.maximum(py1, gy1), 0.0)
                    inter = iw * ih
                    union = parea + garea - inter
                    iou = inter / (union + 1e-8)
                    ew = jnp.maximum(jnp.maximum(px2, gx2)
                                     - jnp.minimum(px1, gx1), 0.0)
                    eh = jnp.maximum(jnp.maximum(py2, gy2)
                                     - jnp.minimum(py1, gy1), 0.0)
                    earea = ew * eh
                    giou = iou - (earea - union) / (earea + 1e-8)
                    cost = 5.0 * bbox + (-prob) + 2.0 * (-giou)

                    qidx = iota + (qstart + j * 16)
                    ltm = cost < bmin
                    bmin = jnp.where(ltm, cost, bmin)
                    bidx = jnp.where(ltm, qidx, bidx)

                obmin[pl.ds((b * G + g) * 16, 16)] = bmin
                obidx[pl.ds((b * G + g) * 16, 16)] = bidx
                return carry
            lax.fori_loop(0, G, g_body, 0)

        pltpu.sync_copy(obmin, omin_hbm.at[pl.ds(wid * owords, owords)])
        pltpu.sync_copy(obidx, oidx_hbm.at[pl.ds(wid * owords, owords)])

    return sc_kernel


def kernel(pred_logits, pred_boxes, gt_boxes, gt_labels):
    B, Q, C = pred_logits.shape
    G = gt_labels.shape[1]
    q0 = Q - _NW * _QPW
    qtc = -(-q0 // _QB_TC) * _QB_TC

    logitsT = jnp.transpose(pred_logits, (2, 0, 1))
    pboxesT = jnp.transpose(pred_boxes, (0, 2, 1))

    gcx, gcy, gw, gh = (gt_boxes[..., 0], gt_boxes[..., 1],
                        gt_boxes[..., 2], gt_boxes[..., 3])
    gx1 = gcx - 0.5 * gw
    gy1 = gcy - 0.5 * gh
    gx2 = gcx + 0.5 * gw
    gy2 = gcy + 0.5 * gh
    garea = (gx2 - gx1) * (gy2 - gy1)
    labf = gt_labels.astype(jnp.float32)
    zero = jnp.zeros_like(gcx)
    gtrec = jnp.stack([gcx, gcy, gw, gh, gx1, gy1, gx2, gy2, garea, labf,
                       zero, zero, zero, zero, zero, zero],
                      axis=2).reshape(-1)

    qsc = _NW * _QPW
    sc = _make_sc_matcher(B, qsc, C, G, q0, _QPW)
    sc_min, sc_idx = sc(logitsT[:, :, q0:].reshape(-1),
                        pboxesT[:, :, q0:].reshape(-1), gtrec)

    tc_min, tc_idx = _match_tc(logitsT, pboxesT, gt_boxes, gt_labels,
                               qb=_QB_TC, q_limit=qtc)

    sc_min = sc_min.reshape(_NW, B * G * 16)
    sc_idx = sc_idx.reshape(_NW, B * G * 16)
    m1 = jnp.min(sc_min, axis=0)
    i1 = jnp.min(jnp.where(sc_min == m1[None], sc_idx, _BIG), axis=0)
    m2 = jnp.min(m1.reshape(B * G, 16), axis=1)
    i2 = jnp.min(jnp.where(m1.reshape(B * G, 16) == m2[:, None],
                           i1.reshape(B * G, 16), _BIG), axis=1)
    sgmin = m2.reshape(B, G)
    sgidx = i2.reshape(B, G)

    all_min = jnp.stack([tc_min, sgmin], axis=0)
    all_idx = jnp.stack([tc_idx, sgidx], axis=0)
    gmin = jnp.min(all_min, axis=0)
    gidx = jnp.min(jnp.where(all_min == gmin[None], all_idx, _BIG), axis=0)
    return gmin, gidx.astype(jnp.int32)
